# baseline (speedup 1.0000x reference)
import jax
import jax.numpy as jnp
from jax.experimental import pallas as pl
from jax.experimental.pallas import tpu as pltpu

_B, _S, _H = 4, 4096, 1024
_F, _T = 4, 8
_OUT = 1024
_FP = 8
_SB = 4096
_NS = _S // _SB


def _body(tok_ref, hid_ref, w_ref, b_ref, out_ref, acc_ref, cacc_ref):
    bi = pl.program_id(0)
    si = pl.program_id(1)

    @pl.when(si == 0)
    def _():
        acc_ref[...] = jnp.zeros_like(acc_ref)
        cacc_ref[...] = jnp.zeros_like(cacc_ref)

    pos = si * _SB + jax.lax.broadcasted_iota(jnp.int32, (1, _SB), 1)
    rows = []
    for f in range(_FP):
        m = jnp.zeros((1, _SB), jnp.bool_)
        if f < _F:
            for t in range(_T):
                st = tok_ref[f, bi, t, 0]
                en = tok_ref[f, bi, t, 1]
                m = m | ((pos >= st) & (pos < en))
        rows.append(m.astype(jnp.float32))
    mask = jnp.concatenate(rows, axis=0)

    h = hid_ref[0]
    acc_ref[...] += jnp.dot(mask, h, preferred_element_type=jnp.float32)
    cacc_ref[...] += jnp.sum(mask, axis=1, keepdims=True)

    @pl.when(si == _NS - 1)
    def _():
        denom = jnp.maximum(cacc_ref[..., 0:1], 1.0)
        pooled = acc_ref[...] / denom
        y = jnp.dot(pooled, w_ref[...], preferred_element_type=jnp.float32)
        y = jnp.tanh(y + b_ref[...])
        out_ref[0] = y[:_F]


_fused = pl.pallas_call(
    _body,
    grid=(_B, _NS),
    in_specs=[
        pl.BlockSpec(memory_space=pltpu.SMEM),
        pl.BlockSpec((1, _SB, _H), lambda b, s: (b, s, 0)),
        pl.BlockSpec((_H, _OUT), lambda b, s: (0, 0)),
        pl.BlockSpec((1, _OUT), lambda b, s: (0, 0)),
    ],
    out_specs=pl.BlockSpec((1, _F, _OUT), lambda b, s: (b, 0, 0)),
    out_shape=jax.ShapeDtypeStruct((_B, _F, _OUT), jnp.float32),
    scratch_shapes=[
        pltpu.VMEM((_FP, _H), jnp.float32),
        pltpu.VMEM((_FP, 128), jnp.float32),
    ],
    compiler_params=pltpu.CompilerParams(
        dimension_semantics=("parallel", "arbitrary"),
    ),
)


def kernel(hidden, token_idxs, W, b):
    tok = token_idxs.astype(jnp.int32)
    return _fused(tok, hidden, W, b.reshape(1, _OUT))
